# Initial kernel scaffold; baseline (speedup 1.0000x reference)
import jax
import jax.numpy as jnp
from jax.experimental import pallas as pl


def kernel(q, k, v):
    raise NotImplementedError("write your pallas kernel here")


# speedup vs baseline: 2.0561x; 2.0561x over previous
import functools

import jax
import jax.numpy as jnp
import numpy as np
from jax.experimental import pallas as pl
from jax.experimental.pallas import tpu as pltpu

NHEADS = 12
HEAD_DIM = 64
BQ = 512


def _attn_kernel(q_ref, k_ref, v_ref, o_ref):
    scale = 1.0 / np.sqrt(HEAD_DIM)
    q = q_ref[0]
    k = k_ref[0]
    v = v_ref[0]
    for h in range(NHEADS):
        lo = h * HEAD_DIM
        qh = q[:, lo:lo + HEAD_DIM]
        kh = k[:, lo:lo + HEAD_DIM]
        vh = v[:, lo:lo + HEAD_DIM]
        s = jax.lax.dot_general(
            qh, kh, (((1,), (1,)), ((), ())),
            preferred_element_type=jnp.float32) * scale
        m = jnp.max(s, axis=-1, keepdims=True)
        e = jnp.exp(s - m)
        p = e / jnp.sum(e, axis=-1, keepdims=True)
        o = jax.lax.dot_general(
            p, vh, (((1,), (0,)), ((), ())),
            preferred_element_type=jnp.float32)
        o_ref[0, :, lo:lo + HEAD_DIM] = o


@jax.jit
def kernel(q, k, v):
    b, s, hd = q.shape
    grid = (b, s // BQ)
    return pl.pallas_call(
        _attn_kernel,
        grid=grid,
        in_specs=[
            pl.BlockSpec((1, BQ, hd), lambda ib, iq: (ib, iq, 0)),
            pl.BlockSpec((1, s, hd), lambda ib, iq: (ib, 0, 0)),
            pl.BlockSpec((1, s, hd), lambda ib, iq: (ib, 0, 0)),
        ],
        out_specs=pl.BlockSpec((1, BQ, hd), lambda ib, iq: (ib, iq, 0)),
        out_shape=jax.ShapeDtypeStruct((b, s, hd), q.dtype),
        compiler_params=pltpu.CompilerParams(
            dimension_semantics=("parallel", "arbitrary"),
        ),
    )(q, k, v)


# speedup vs baseline: 2.3075x; 1.1223x over previous
import functools

import jax
import jax.numpy as jnp
import numpy as np
from jax.experimental import pallas as pl
from jax.experimental.pallas import tpu as pltpu

NHEADS = 12
HEAD_DIM = 64
BQ = 256


def _attn_kernel(q_ref, k_ref, v_ref, o_ref):
    scale = 1.0 / np.sqrt(HEAD_DIM)
    q = q_ref[0]
    k = k_ref[0]
    v = v_ref[0]
    for h in range(NHEADS):
        lo = h * HEAD_DIM
        qh = (q[:, lo:lo + HEAD_DIM] * scale).astype(jnp.bfloat16)
        kh = k[:, lo:lo + HEAD_DIM].astype(jnp.bfloat16)
        vh = v[:, lo:lo + HEAD_DIM].astype(jnp.bfloat16)
        s = jax.lax.dot_general(
            qh, kh, (((1,), (1,)), ((), ())),
            preferred_element_type=jnp.float32)
        m = jnp.max(s, axis=-1, keepdims=True)
        e = jnp.exp(s - m)
        r = 1.0 / jnp.sum(e, axis=-1, keepdims=True)
        p = e.astype(jnp.bfloat16)
        o = jax.lax.dot_general(
            p, vh, (((1,), (0,)), ((), ())),
            preferred_element_type=jnp.float32)
        o_ref[0, :, lo:lo + HEAD_DIM] = o * r


@jax.jit
def kernel(q, k, v):
    b, s, hd = q.shape
    grid = (b, s // BQ)
    return pl.pallas_call(
        _attn_kernel,
        grid=grid,
        in_specs=[
            pl.BlockSpec((1, BQ, hd), lambda ib, iq: (ib, iq, 0)),
            pl.BlockSpec((1, s, hd), lambda ib, iq: (ib, 0, 0)),
            pl.BlockSpec((1, s, hd), lambda ib, iq: (ib, 0, 0)),
        ],
        out_specs=pl.BlockSpec((1, BQ, hd), lambda ib, iq: (ib, iq, 0)),
        out_shape=jax.ShapeDtypeStruct((b, s, hd), q.dtype),
        compiler_params=pltpu.CompilerParams(
            dimension_semantics=("parallel", "arbitrary"),
            vmem_limit_bytes=100 * 1024 * 1024,
        ),
    )(q, k, v)


# speedup vs baseline: 2.3503x; 1.0185x over previous
import functools

import jax
import jax.numpy as jnp
import numpy as np
from jax.experimental import pallas as pl
from jax.experimental.pallas import tpu as pltpu

NHEADS = 12
HEAD_DIM = 64
BQ = 256


def _attn_kernel(q_ref, k_ref, v_ref, o_ref):
    scale = np.log2(np.e) / np.sqrt(HEAD_DIM)
    q = q_ref[0]
    k = k_ref[0]
    v = v_ref[0]
    for h in range(NHEADS):
        lo = h * HEAD_DIM
        qh = (q[:, lo:lo + HEAD_DIM] * scale).astype(jnp.bfloat16)
        kh = k[:, lo:lo + HEAD_DIM].astype(jnp.bfloat16)
        vh = v[:, lo:lo + HEAD_DIM].astype(jnp.bfloat16)
        s = jax.lax.dot_general(
            qh, kh, (((1,), (1,)), ((), ())),
            preferred_element_type=jnp.float32)
        m = jnp.max(s, axis=-1, keepdims=True)
        e = jnp.exp2(s - m)
        r = 1.0 / jnp.sum(e, axis=-1, keepdims=True)
        p = e.astype(jnp.bfloat16)
        o = jax.lax.dot_general(
            p, vh, (((1,), (0,)), ((), ())),
            preferred_element_type=jnp.float32)
        o_ref[0, :, lo:lo + HEAD_DIM] = o * r


@jax.jit
def kernel(q, k, v):
    b, s, hd = q.shape
    grid = (b, s // BQ)
    return pl.pallas_call(
        _attn_kernel,
        grid=grid,
        in_specs=[
            pl.BlockSpec((1, BQ, hd), lambda ib, iq: (ib, iq, 0)),
            pl.BlockSpec((1, s, hd), lambda ib, iq: (ib, 0, 0)),
            pl.BlockSpec((1, s, hd), lambda ib, iq: (ib, 0, 0)),
        ],
        out_specs=pl.BlockSpec((1, BQ, hd), lambda ib, iq: (ib, iq, 0)),
        out_shape=jax.ShapeDtypeStruct((b, s, hd), q.dtype),
        compiler_params=pltpu.CompilerParams(
            dimension_semantics=("parallel", "arbitrary"),
            vmem_limit_bytes=100 * 1024 * 1024,
        ),
    )(q, k, v)
